# baseline (speedup 1.0000x reference)
import functools

import jax
import jax.numpy as jnp
from jax import lax
from jax.experimental import pallas as pl
from jax.experimental.pallas import tpu as pltpu
from jax.experimental.pallas import tpu_sc as plsc

N_NODES = 10000
D_FEAT = 128
UNITS = 128
HALF_UNITS = UNITS // 2
HALF_N = N_NODES // 2
ACC_ROWS = HALF_N + 8

LANES = 16
CHUNK = 96
N_TILES = 16
SUP_PACK_ROWS = N_NODES // 2
SUP_ROWS_PER_TILE = 312
SUP_TAIL = SUP_PACK_ROWS - 16 * SUP_ROWS_PER_TILE
ACC_ROWS_PER_TILE = 312
ACC_TAIL = ACC_ROWS - 16 * ACC_ROWS_PER_TILE


def _mm_body(x_ref, w_ref, o_ref):
    o_ref[...] = jnp.dot(x_ref[...], w_ref[...],
                         preferred_element_type=jnp.float32
                         ).astype(jnp.bfloat16)


def _matmul_perm(x, w_perm):
    m = x.shape[0]
    blk = 2000
    return pl.pallas_call(
        _mm_body,
        grid=(m // blk,),
        in_specs=[
            pl.BlockSpec((blk, D_FEAT), lambda i: (i, 0)),
            pl.BlockSpec((D_FEAT, UNITS), lambda i: (0, 0)),
        ],
        out_specs=pl.BlockSpec((blk, UNITS), lambda i: (i, 0)),
        out_shape=jax.ShapeDtypeStruct((m, UNITS), jnp.bfloat16),
    )(x, w_perm)


def _combine_body(p_ref, b_ref, o_ref):
    o_ref[...] = p_ref[0] + b_ref[...]


def _combine(parts, b2d):
    blk = 1000
    nb = HALF_N // blk
    return pl.pallas_call(
        _combine_body,
        grid=(2 * nb,),
        in_specs=[
            pl.BlockSpec((1, blk, UNITS), lambda i: (i // nb, i % nb, 0)),
            pl.BlockSpec((1, UNITS), lambda i: (0, 0)),
        ],
        out_specs=pl.BlockSpec((blk, UNITS), lambda i: (i, 0)),
        out_shape=jax.ShapeDtypeStruct((N_NODES, UNITS), jnp.float32),
    )(parts, b2d)


def _make_sc_kernel(n_chunks):
    assert n_chunks % 2 == 0 and n_chunks >= 4
    mesh = plsc.VectorSubcoreMesh(core_axis_name="c", subcore_axis_name="s")

    @functools.partial(
        pl.kernel,
        mesh=mesh,
        out_type=jax.ShapeDtypeStruct((2, ACC_ROWS, UNITS), jnp.float32),
        scratch_types=[
            pltpu.VMEM((2, CHUNK), jnp.int32),
            pltpu.VMEM((2, CHUNK), jnp.int32),
            pltpu.VMEM((2, CHUNK), jnp.int32),
            pltpu.VMEM((2, CHUNK), jnp.int32),
            pltpu.VMEM((2, CHUNK), jnp.float32),
            pltpu.VMEM((2, CHUNK, UNITS), jnp.int32),
            pltpu.VMEM((CHUNK, UNITS), jnp.float32),
            pltpu.VMEM_SHARED((SUP_PACK_ROWS, UNITS), jnp.int32),
            pltpu.VMEM_SHARED((ACC_ROWS, UNITS), jnp.float32),
            pltpu.SemaphoreType.DMA,
            pltpu.SemaphoreType.DMA,
            pltpu.SemaphoreType.DMA,
            pltpu.SemaphoreType.DMA,
            pltpu.SemaphoreType.DMA,
            pltpu.SemaphoreType.DMA,
        ],
    )
    def sc_kernel(sup_hbm, packed_hbm, w_hbm, zeros_hbm, out_hbm,
                  pbuf, src_r, dst_r, off_r, w_r, gbuf, sbuf, sup_sh, acc,
                  gsem0, gsem1, wsem0, wsem1, psem0, psem1):
        cid = lax.axis_index("c")
        sid = lax.axis_index("s")
        gsem = (gsem0, gsem1)
        wsem = (wsem0, wsem1)
        psem = (psem0, psem1)
        base = cid * HALF_N

        srow0 = sid * SUP_ROWS_PER_TILE
        pltpu.sync_copy(sup_hbm.at[pl.ds(srow0, SUP_ROWS_PER_TILE)],
                        sup_sh.at[pl.ds(srow0, SUP_ROWS_PER_TILE)])
        arow0 = sid * ACC_ROWS_PER_TILE
        pltpu.sync_copy(zeros_hbm.at[pl.ds(arow0, ACC_ROWS_PER_TILE)],
                        acc.at[pl.ds(arow0, ACC_ROWS_PER_TILE)])

        @pl.when(sid == 0)
        def _():
            stail = 16 * SUP_ROWS_PER_TILE
            pltpu.sync_copy(sup_hbm.at[pl.ds(stail, SUP_TAIL)],
                            sup_sh.at[pl.ds(stail, SUP_TAIL)])
            atail = 16 * ACC_ROWS_PER_TILE
            pltpu.sync_copy(zeros_hbm.at[pl.ds(atail, ACC_TAIL)],
                            acc.at[pl.ds(atail, ACC_TAIL)])

        plsc.subcore_barrier()

        def issue_pload(j, b):
            pltpu.async_copy(packed_hbm.at[sid, j], pbuf.at[b], psem[b])

        def wait_pload(b):
            pltpu.make_async_copy(packed_hbm.at[0, 0], pbuf.at[b],
                                  psem[b]).wait()

        def unpack_idx(b):
            for g in range(CHUNK // LANES):
                sl = pl.ds(g * LANES, LANES)
                p = pbuf[b, sl]
                sidx = p & 0xFFFF
                src_r[b, sl] = sidx >> 1
                off_r[b, sl] = (sidx & 1) * HALF_UNITS
                d = (p >> 16) - base
                ok = (d >= 0) & (d < HALF_N)
                dst_r[b, sl] = jnp.where(ok, d, HALF_N)

        def issue_wload(j, b):
            pltpu.async_copy(w_hbm.at[sid, j], w_r.at[b], wsem[b])

        def wait_wload(b):
            pltpu.make_async_copy(w_hbm.at[0, 0], w_r.at[b], wsem[b]).wait()

        def issue_gather(b):
            pltpu.async_copy(sup_sh.at[src_r.at[b]], gbuf.at[b], gsem[b])

        def wait_gather(b):
            pltpu.make_async_copy(sup_sh.at[pl.ds(0, CHUNK)],
                                  gbuf.at[b], gsem[b]).wait()

        def do_mul(b):
            gbi = gbuf.at[b]
            sbi = sbuf.bitcast(jnp.int32)

            def widen_body(g, c2):
                offg = off_r[b, pl.ds(g * LANES, LANES)]
                for l in range(LANES):
                    e = g * LANES + l
                    off = offg[l]
                    for q in range(UNITS // (2 * LANES)):
                        words = gbi[e, pl.ds(off + q * LANES, LANES)]
                        sbi[e, pl.ds(q * LANES, LANES)] = words << 16
                        sbi[e, pl.ds(HALF_UNITS + q * LANES, LANES)] = (
                            words & jnp.int32(-65536))
                return c2

            lax.fori_loop(0, CHUNK // LANES, widen_body, 0)

            def scale_body(g, c2):
                wg = w_r[b, pl.ds(g * LANES, LANES)]
                for l in range(LANES):
                    e = g * LANES + l
                    wvec = jnp.full((LANES,), wg[l], dtype=jnp.float32)
                    for c in range(UNITS // LANES):
                        sl = pl.ds(c * LANES, LANES)
                        sbuf[e, sl] = sbuf[e, sl] * wvec
                return c2

            lax.fori_loop(0, CHUNK // LANES, scale_body, 0)

        for b in range(2):
            issue_pload(b, b)
        for b in range(2):
            wait_pload(b)
            unpack_idx(b)
            issue_pload(b + 2, b)
            issue_wload(b, b)
            issue_gather(b)

        def pair_body(jj, carry):
            j0 = 2 * jj
            for b in range(2):
                j = j0 + b
                wait_gather(b)
                wait_wload(b)
                do_mul(b)
                pltpu.sync_copy(sbuf, acc.at[dst_r.at[b]], add=True)
                wait_pload(b)
                unpack_idx(b)
                issue_pload(jnp.minimum(j + 4, n_chunks - 1), b)
                issue_wload(j + 2, b)
                issue_gather(b)
            return carry

        lax.fori_loop(0, (n_chunks - 2) // 2, pair_body, 0)

        for b in range(2):
            wait_gather(b)
            wait_wload(b)
            do_mul(b)
            pltpu.sync_copy(sbuf, acc.at[dst_r.at[b]], add=True)
        for b in range(2):
            wait_pload(b)

        plsc.subcore_barrier()

        pltpu.sync_copy(acc.at[pl.ds(arow0, ACC_ROWS_PER_TILE)],
                        out_hbm.at[cid, pl.ds(arow0, ACC_ROWS_PER_TILE)])

        @pl.when(sid == 0)
        def _():
            atail = 16 * ACC_ROWS_PER_TILE
            pltpu.sync_copy(acc.at[pl.ds(atail, ACC_TAIL)],
                            out_hbm.at[cid, pl.ds(atail, ACC_TAIL)])

    return sc_kernel


@jax.jit
def kernel(inputs, edge_index, edge_weight, W, b):
    n_edges = edge_index.shape[1]
    n_chunks = -(-n_edges // (N_TILES * CHUNK))
    n_chunks = max(4, n_chunks + (n_chunks % 2))
    padded = N_TILES * n_chunks * CHUNK
    pad = padded - n_edges

    packed = jnp.pad(edge_index[0] | (edge_index[1] << 16), (0, pad))
    packed = packed.reshape(N_TILES, n_chunks, CHUNK)
    w = jnp.pad(edge_weight, (0, pad)).reshape(N_TILES, n_chunks, CHUNK)

    sigma = jnp.arange(UNITS) // 2 + (jnp.arange(UNITS) % 2) * HALF_UNITS
    sup_bf = _matmul_perm(inputs, W[:, sigma])
    sup = jax.lax.bitcast_convert_type(
        sup_bf.reshape(SUP_PACK_ROWS, UNITS, 2), jnp.int32)
    zeros = jnp.zeros((ACC_ROWS, UNITS), jnp.float32)
    parts = _make_sc_kernel(n_chunks)(sup, packed, w, zeros)
    return _combine(parts, b.reshape(1, UNITS))


# speedup vs baseline: 2.4972x; 2.4972x over previous
import functools

import jax
import jax.numpy as jnp
from jax import lax
from jax.experimental import pallas as pl
from jax.experimental.pallas import tpu as pltpu
from jax.experimental.pallas import tpu_sc as plsc

N_NODES = 10000
D_FEAT = 128
UNITS = 128

LANES = 16
CHUNK = 128
N_WORKERS = 32
ROWS_PER_TILE = 624
TAIL_ROWS = N_NODES - 16 * ROWS_PER_TILE


def _mm_body(x_ref, w_ref, o_ref):
    o_ref[...] = jnp.dot(x_ref[...], w_ref[...],
                         preferred_element_type=jnp.float32)


def _matmul(x, w):
    m = x.shape[0]
    blk = 1000
    grid = m // blk
    return pl.pallas_call(
        _mm_body,
        grid=(grid,),
        in_specs=[
            pl.BlockSpec((blk, D_FEAT), lambda i: (i, 0)),
            pl.BlockSpec((D_FEAT, UNITS), lambda i: (0, 0)),
        ],
        out_specs=pl.BlockSpec((blk, UNITS), lambda i: (i, 0)),
        out_shape=jax.ShapeDtypeStruct((m, UNITS), jnp.float32),
    )(x, w)


def _combine_body(p0_ref, p1_ref, b_ref, o_ref):
    o_ref[...] = p0_ref[...] + p1_ref[...] + b_ref[...]


def _combine(p0, p1, b2d):
    m = p0.shape[0]
    blk = 1000
    grid = m // blk
    return pl.pallas_call(
        _combine_body,
        grid=(grid,),
        in_specs=[
            pl.BlockSpec((blk, UNITS), lambda i: (i, 0)),
            pl.BlockSpec((blk, UNITS), lambda i: (i, 0)),
            pl.BlockSpec((1, UNITS), lambda i: (0, 0)),
        ],
        out_specs=pl.BlockSpec((blk, UNITS), lambda i: (i, 0)),
        out_shape=jax.ShapeDtypeStruct((m, UNITS), jnp.float32),
    )(p0, p1, b2d)


def _make_sc_kernel(n_chunks):
    mesh = plsc.VectorSubcoreMesh(core_axis_name="c", subcore_axis_name="s")

    @functools.partial(
        pl.kernel,
        mesh=mesh,
        out_type=jax.ShapeDtypeStruct((2, N_NODES, UNITS), jnp.float32),
        scratch_types=[
            pltpu.VMEM((n_chunks, CHUNK), jnp.int32),
            pltpu.VMEM((n_chunks, CHUNK), jnp.int32),
            pltpu.VMEM((n_chunks, CHUNK), jnp.float32),
            pltpu.VMEM((CHUNK, UNITS), jnp.float32),
            pltpu.VMEM_SHARED((N_NODES, UNITS), jnp.float32),
            pltpu.SemaphoreType.DMA,
        ],
    )
    def sc_kernel(support_hbm, src_hbm, dst_hbm, w_hbm, zeros_hbm, out_hbm,
                  src_v, dst_v, w_v, buf, acc, sem):
        cid = lax.axis_index("c")
        sid = lax.axis_index("s")
        wid = cid * 16 + sid

        row0 = sid * ROWS_PER_TILE
        pltpu.sync_copy(zeros_hbm.at[pl.ds(row0, ROWS_PER_TILE)],
                        acc.at[pl.ds(row0, ROWS_PER_TILE)])

        @pl.when(sid == 0)
        def _():
            tail0 = 16 * ROWS_PER_TILE
            pltpu.sync_copy(zeros_hbm.at[pl.ds(tail0, TAIL_ROWS)],
                            acc.at[pl.ds(tail0, TAIL_ROWS)])

        pltpu.sync_copy(src_hbm.at[wid], src_v)
        pltpu.sync_copy(dst_hbm.at[wid], dst_v)
        pltpu.sync_copy(w_hbm.at[wid], w_v)

        plsc.subcore_barrier()

        def chunk_body(j, carry):
            pltpu.async_copy(support_hbm.at[src_v.at[j]], buf, sem).wait()

            def group_body(g, c2):
                wg = w_v[j, pl.ds(g * LANES, LANES)]
                for l in range(LANES):
                    e = g * LANES + l
                    wvec = jnp.full((LANES,), wg[l], dtype=jnp.float32)
                    for c in range(UNITS // LANES):
                        sl = pl.ds(c * LANES, LANES)
                        buf[e, sl] = buf[e, sl] * wvec
                return c2

            lax.fori_loop(0, CHUNK // LANES, group_body, 0)

            pltpu.sync_copy(buf, acc.at[dst_v.at[j]], add=True)
            return carry

        lax.fori_loop(0, n_chunks, chunk_body, 0)

        plsc.subcore_barrier()

        pltpu.sync_copy(acc.at[pl.ds(row0, ROWS_PER_TILE)],
                        out_hbm.at[cid, pl.ds(row0, ROWS_PER_TILE)])

        @pl.when(sid == 0)
        def _():
            tail0 = 16 * ROWS_PER_TILE
            pltpu.sync_copy(acc.at[pl.ds(tail0, TAIL_ROWS)],
                            out_hbm.at[cid, pl.ds(tail0, TAIL_ROWS)])

    return sc_kernel


@jax.jit
def kernel(inputs, edge_index, edge_weight, W, b):
    n_edges = edge_index.shape[1]
    n_chunks = -(-n_edges // (N_WORKERS * CHUNK))
    padded = N_WORKERS * n_chunks * CHUNK
    pad = padded - n_edges

    src = jnp.pad(edge_index[0], (0, pad)).reshape(N_WORKERS, n_chunks, CHUNK)
    dst = jnp.pad(edge_index[1], (0, pad)).reshape(N_WORKERS, n_chunks, CHUNK)
    w = jnp.pad(edge_weight, (0, pad)).reshape(N_WORKERS, n_chunks, CHUNK)

    support = _matmul(inputs, W)
    zeros = jnp.zeros((N_NODES, UNITS), jnp.float32)
    parts = _make_sc_kernel(n_chunks)(support, src, dst, w, zeros)
    return _combine(parts[0], parts[1], b.reshape(1, UNITS))
